# baseline (speedup 1.0000x reference)
import functools

import jax
import jax.numpy as jnp
from jax import lax
from jax.experimental import pallas as pl
from jax.experimental.pallas import tpu as pltpu
from jax.experimental.pallas import tpu_sc as plsc

R = 64
N = 4096
L = 16
C = N // L
NC = 2
NS = 16
NW = NC * NS
RPW = R // NW


def _sc_body(z_hbm, u_hbm, o_hbm, zv, uv, ev):
    wid = lax.axis_index("s") * NC + lax.axis_index("c")
    base = wid * RPW
    pltpu.sync_copy(z_hbm.at[pl.ds(base, RPW)], zv)
    pltpu.sync_copy(u_hbm.at[pl.ds(base, RPW)], uv)

    zeros = jnp.zeros((L,), jnp.float32)
    neg_inf = jnp.float32(-jnp.inf)

    for r in range(RPW):
        def max_body(i, acc, r=r):
            z16 = zv[r, pl.ds(i * L, L)]
            u16 = uv[r, pl.ds(i * L, L)]
            return jnp.maximum(acc, jnp.where(u16 > 0.0, z16, neg_inf))

        mx = lax.fori_loop(0, C, max_body, jnp.full((L,), neg_inf, jnp.float32))
        zmax = jnp.max(mx, axis=0)

        def exp_body(i, acc, r=r):
            z16 = zv[r, pl.ds(i * L, L)]
            u16 = uv[r, pl.ds(i * L, L)]
            e16 = jnp.where(u16 > 0.0, jnp.exp(z16 - zmax), 0.0)
            ev[r, pl.ds(i * L, L)] = e16
            return acc + e16

        Z = jnp.sum(lax.fori_loop(0, C, exp_body, zeros), axis=0)

        def fp_step(state, r=r):
            tau_v, E_prev, U_prev, k = state

            def accum(i, accs, r=r, tau_v=tau_v):
                aE, aU = accs
                e16 = ev[r, pl.ds(i * L, L)]
                u16 = uv[r, pl.ds(i * L, L)]
                sat = e16 > tau_v * u16
                return (aE + jnp.where(sat, e16, 0.0),
                        aU + jnp.where(sat, u16, 0.0))

            aE, aU = lax.fori_loop(0, C, accum, (zeros, zeros))
            E = jnp.sum(aE, axis=0)
            U = jnp.sum(aU, axis=0)
            num = jnp.maximum(Z - E, 0.0)
            den = jnp.maximum(1.0 - U, 1e-30)
            new_tau_v = jnp.full((L,), num) / jnp.full((L,), den)
            return new_tau_v, E, U, k + 1

        def fp_cond2(state):
            tau_v, EU, k = state
            E, U, E_prev, U_prev = EU
            changed = jnp.logical_or(E != E_prev, U != U_prev)
            return jnp.logical_and(k < 64, changed)

        def fp_step2(state, r=r):
            tau_v, EU, k = state
            E_old, U_old = EU[0], EU[1]
            new_tau_v, E, U, k2 = fp_step((tau_v, E_old, U_old, k))
            return new_tau_v, (E, U, E_old, U_old), k2

        tau0_v = jnp.full((L,), Z)
        st = fp_step2((tau0_v, (jnp.float32(-1.0), jnp.float32(-1.0),
                                jnp.float32(-2.0), jnp.float32(-2.0)),
                       jnp.int32(0)))
        tau_v, _, _ = lax.while_loop(fp_cond2, fp_step2, st)
        tau_v = jnp.maximum(tau_v, jnp.float32(1e-30))

        def out_body(i, carry, r=r, tau_v=tau_v):
            e16 = ev[r, pl.ds(i * L, L)]
            u16 = uv[r, pl.ds(i * L, L)]
            ev[r, pl.ds(i * L, L)] = jnp.where(
                u16 > 0.0, jnp.minimum(u16, e16 / tau_v), 0.0)
            return carry

        lax.fori_loop(0, C, out_body, jnp.int32(0))

    pltpu.sync_copy(ev, o_hbm.at[pl.ds(base, RPW)])


@jax.jit
def kernel(input1, input2):
    mesh = plsc.VectorSubcoreMesh(
        core_axis_name="c", subcore_axis_name="s",
        num_cores=NC, num_subcores=NS)
    return pl.kernel(
        _sc_body,
        out_type=jax.ShapeDtypeStruct((R, N), jnp.float32),
        mesh=mesh,
        compiler_params=pltpu.CompilerParams(needs_layout_passes=False),
        scratch_types=[
            pltpu.VMEM((RPW, N), jnp.float32),
            pltpu.VMEM((RPW, N), jnp.float32),
            pltpu.VMEM((RPW, N), jnp.float32),
        ],
    )(input1, input2)


# speedup vs baseline: 1.3203x; 1.3203x over previous
import jax
import jax.numpy as jnp
from jax import lax
from jax.experimental import pallas as pl
from jax.experimental.pallas import tpu as pltpu
from jax.experimental.pallas import tpu_sc as plsc

R = 64
N = 4096
L = 16
NC = 2
NS = 16
NW = NC * NS
RPW = R // NW


def _splat(x):
    return jnp.full((L,), x, dtype=jnp.float32)


def _tau_vec(Z, E, U):
    num = jnp.maximum(Z - E, 0.0)
    den = jnp.maximum(1.0 - U, 1e-30)
    return _splat(num) / _splat(den)


def _sc_body(z_hbm, u_hbm, o_hbm, zv, uv, ev, ov):
    wid = lax.axis_index("s") * NC + lax.axis_index("c")
    base = wid * RPW
    pltpu.sync_copy(z_hbm.at[pl.ds(base, RPW)], zv)
    pltpu.sync_copy(u_hbm.at[pl.ds(base, RPW)], uv)

    zeros = jnp.zeros((L,), jnp.float32)
    neg_inf_v = _splat(-jnp.inf)

    @plsc.parallel_loop(0, N, L, unroll=4, carry=(neg_inf_v, neg_inf_v))
    def p1(i, ms):
        m0, m1 = ms
        return (jnp.maximum(m0, zv[0, pl.ds(i, L)]),
                jnp.maximum(m1, zv[1, pl.ds(i, L)]))

    M0 = jnp.max(p1[0], axis=0)
    M1 = jnp.max(p1[1], axis=0)
    M0v, M1v = _splat(M0), _splat(M1)

    @plsc.parallel_loop(0, N, L, unroll=4, carry=(zeros, zeros))
    def p2(i, ss):
        s0, s1 = ss
        sl = pl.ds(i, L)
        e0 = jnp.where(uv[0, sl] > 0.0, jnp.exp(zv[0, sl] - M0v), 0.0)
        e1 = jnp.where(uv[1, sl] > 0.0, jnp.exp(zv[1, sl] - M1v), 0.0)
        ev[0, sl] = e0
        ev[1, sl] = e1
        return (s0 + e0, s1 + e1)

    Z0 = jnp.sum(p2[0], axis=0)
    Z1 = jnp.sum(p2[1], axis=0)
    t0 = _splat(Z0)
    t1 = _splat(Z1)

    @plsc.parallel_loop(0, N, L, unroll=4,
                        carry=(zeros, zeros, zeros, zeros))
    def p3(i, accs):
        aE0, aU0, aE1, aU1 = accs
        sl = pl.ds(i, L)
        e0, u0 = ev[0, sl], uv[0, sl]
        e1, u1 = ev[1, sl], uv[1, sl]
        s0 = e0 > t0 * u0
        s1 = e1 > t1 * u1
        return (aE0 + jnp.where(s0, e0, 0.0), aU0 + jnp.where(s0, u0, 0.0),
                aE1 + jnp.where(s1, e1, 0.0), aU1 + jnp.where(s1, u1, 0.0))

    E0 = jnp.sum(p3[0], axis=0)
    U0 = jnp.sum(p3[1], axis=0)
    E1 = jnp.sum(p3[2], axis=0)
    U1 = jnp.sum(p3[3], axis=0)
    tau0 = _tau_vec(Z0, E0, U0)
    tau1 = _tau_vec(Z1, E1, U1)
    inv0 = _splat(1.0) / jnp.maximum(tau0, 1e-30)
    inv1 = _splat(1.0) / jnp.maximum(tau1, 1e-30)

    @plsc.parallel_loop(0, N, L, unroll=4,
                        carry=(zeros, zeros, zeros, zeros))
    def p4(i, accs):
        aE0, aU0, aE1, aU1 = accs
        sl = pl.ds(i, L)
        e0, u0 = ev[0, sl], uv[0, sl]
        e1, u1 = ev[1, sl], uv[1, sl]
        s0 = e0 > tau0 * u0
        s1 = e1 > tau1 * u1
        ov[0, sl] = jnp.where(u0 > 0.0, jnp.minimum(u0, e0 * inv0), 0.0)
        ov[1, sl] = jnp.where(u1 > 0.0, jnp.minimum(u1, e1 * inv1), 0.0)
        return (aE0 + jnp.where(s0, e0, 0.0), aU0 + jnp.where(s0, u0, 0.0),
                aE1 + jnp.where(s1, e1, 0.0), aU1 + jnp.where(s1, u1, 0.0))

    for r, (Z, E_1, U_1, acc_idx) in enumerate(((Z0, E0, U0, 0),
                                                (Z1, E1, U1, 2))):
        E_2 = jnp.sum(p4[acc_idx], axis=0)
        U_2 = jnp.sum(p4[acc_idx + 1], axis=0)

        def fp_cond(state):
            E_new, U_new, E_old, U_old, k = state
            changed = jnp.logical_or(E_new != E_old, U_new != U_old)
            return jnp.logical_and(k < 64, changed)

        def fp_step(state, r=r, Z=Z):
            E_new, U_new, _, _, k = state
            tau = _tau_vec(Z, E_new, U_new)
            inv = _splat(1.0) / jnp.maximum(tau, 1e-30)

            def accum(i, accs, r=r, tau=tau, inv=inv):
                aE, aU = accs
                sl = pl.ds(i * L, L)
                e16, u16 = ev[r, sl], uv[r, sl]
                sat = e16 > tau * u16
                ov[r, sl] = jnp.where(
                    u16 > 0.0, jnp.minimum(u16, e16 * inv), 0.0)
                return (aE + jnp.where(sat, e16, 0.0),
                        aU + jnp.where(sat, u16, 0.0))

            aE, aU = lax.fori_loop(0, N // L, accum, (zeros, zeros))
            return (jnp.sum(aE, axis=0), jnp.sum(aU, axis=0),
                    E_new, U_new, k + 1)

        lax.while_loop(fp_cond, fp_step,
                       (E_2, U_2, E_1, U_1, jnp.int32(0)))

    pltpu.sync_copy(ov, o_hbm.at[pl.ds(base, RPW)])


@jax.jit
def kernel(input1, input2):
    mesh = plsc.VectorSubcoreMesh(
        core_axis_name="c", subcore_axis_name="s",
        num_cores=NC, num_subcores=NS)
    return pl.kernel(
        _sc_body,
        out_type=jax.ShapeDtypeStruct((R, N), jnp.float32),
        mesh=mesh,
        compiler_params=pltpu.CompilerParams(needs_layout_passes=False),
        scratch_types=[
            pltpu.VMEM((RPW, N), jnp.float32),
            pltpu.VMEM((RPW, N), jnp.float32),
            pltpu.VMEM((RPW, N), jnp.float32),
            pltpu.VMEM((RPW, N), jnp.float32),
        ],
    )(input1, input2)
